# baseline (speedup 1.0000x reference)
import functools

import jax
from jax import lax
import jax.numpy as jnp
from jax.experimental import pallas as pl
from jax.experimental.pallas import tpu as pltpu
from jax.experimental.pallas import tpu_sc as plsc

_BM = 256
_KPAD = 5
_LANES = 16
_NW = 32
_CITEMS = 80
_YPAD = 128


def _topk_idx_w(s, k, kpad, exact_ties):
    bm, n = s.shape
    iota = (jax.lax.broadcasted_iota(jnp.int32, (bm, n), 1)
            if exact_ties else None)
    js, es = [], []
    v0 = None
    m = None
    for step in range(k):
        if exact_ties:
            t = s
        else:
            t = s if step == 0 else jnp.where(s < m, s, -jnp.inf)
        m = jnp.max(t, axis=1, keepdims=True)
        ji = jnp.argmax(t, axis=1)[:, None]
        if step == 0:
            v0 = m
            e = jnp.ones_like(m)
            z = jnp.ones_like(m)
        else:
            e = jnp.exp(m - v0)
            z = z + e
        js.append(ji.astype(jnp.int32))
        es.append(e)
        if exact_ties and step < k - 1:
            s = jnp.where(iota == ji, -jnp.inf, s)
    zi = 1.0 / z
    wcols = [jnp.broadcast_to(e * zi, (bm, _LANES)) for e in es]
    for _ in range(kpad - k):
        js.append(jnp.zeros_like(js[0]))
        wcols.append(jnp.zeros((bm, _LANES), jnp.float32))
    return jnp.concatenate(js, axis=1), jnp.concatenate(wcols, axis=1)


def _tc_body(x_ref, xt_ref, pc_ref, pct_ref,
             sidx_ref, sw_ref, pidx_ref, pw_ref, *, sim_k, spat_k):
    s = jnp.dot(x_ref[...], xt_ref[...], preferred_element_type=jnp.float32)
    sidx_ref[...], sw_ref[...] = _topk_idx_w(s, sim_k, _KPAD,
                                             exact_ties=False)

    pcb = pc_ref[...]
    pct = pct_ref[...]
    c2 = jnp.sum(pct * pct, axis=0, keepdims=True)
    c2b = jnp.sum(pcb * pcb, axis=1, keepdims=True)
    nd = -(c2b + c2
           - 2.0 * jnp.dot(pcb, pct, preferred_element_type=jnp.float32))
    pidx_ref[...], pw_ref[...] = _topk_idx_w(nd, spat_k, _KPAD,
                                             exact_ties=True)


_NBUF = 4


def _sc_gather_kernel(y_hbm, idx_hbm, w_hbm, out_hbm,
                      idx_v, rows_v, w_v, out_v, sems, *, qpw):
    wid = lax.axis_index("s") * 2 + lax.axis_index("c")
    items = qpw * _KPAD
    nchunks = items // _CITEMS
    qpc = _CITEMS // _KPAD
    wrows = _CITEMS // 8
    orows = qpc // 8

    pltpu.sync_copy(idx_hbm.at[wid], idx_v)

    def _fire(c, b):
        pltpu.async_copy(y_hbm.at[idx_v.at[c]], rows_v.at[b], sems.at[b])
        pltpu.async_copy(w_hbm.at[wid * nchunks + c], w_v.at[b], sems.at[b])

    def _wait(b):
        pltpu.make_async_copy(y_hbm.at[pl.ds(0, _CITEMS), :],
                              rows_v.at[b], sems.at[b]).wait()
        pltpu.make_async_copy(w_hbm.at[0], w_v.at[b], sems.at[b]).wait()

    def _compute(c, b):
        for qq in range(qpc):
            j0 = qq * _KPAD
            acc = None
            for k in range(_KPAD):
                j = j0 + k
                w = w_v[b, j // 8, pl.ds((j % 8) * _LANES, _LANES)]
                r = rows_v[b, j, pl.ds(0, _LANES)]
                acc = r * w if acc is None else acc + r * w
            out_v[c * orows + qq // 8,
                  pl.ds((qq % 8) * _LANES, _LANES)] = acc

    for b in range(_NBUF):
        _fire(b, b)

    def body(i, carry):
        for b in range(_NBUF):
            c = i * _NBUF + b
            _wait(b)
            _compute(c, b)
            _fire(c + _NBUF, b)
        return carry

    lax.fori_loop(0, nchunks // _NBUF - 1, body, 0)
    for b in range(_NBUF):
        c = nchunks - _NBUF + b
        _wait(b)
        _compute(c, b)

    opw = qpw * _LANES // 128
    pltpu.sync_copy(out_v, out_hbm.at[pl.ds(wid * opw, opw), :])


_NSLAB = 8


def kernel(x, y, patch_centers):
    n, d_feat = x.shape
    d_y = y.shape[1]
    xt = x.T
    pct = patch_centers.T
    y_pad = jnp.pad(y, ((0, 0), (0, _YPAD - d_y)))
    mesh = plsc.VectorSubcoreMesh(core_axis_name="c", subcore_axis_name="s")

    ns = n // _NSLAB
    q_slab = 2 * ns
    qpw = q_slab // _NW
    items = qpw * _KPAD
    tc_body = functools.partial(_tc_body, sim_k=5, spat_k=4)
    sc_fn = functools.partial(_sc_gather_kernel, qpw=qpw)

    outs = []
    for s in range(_NSLAB):
        xs = jax.lax.slice_in_dim(x, s * ns, (s + 1) * ns)
        pcs = jax.lax.slice_in_dim(patch_centers, s * ns, (s + 1) * ns)
        sidx, sw, pidx, pw = pl.pallas_call(
            tc_body,
            grid=(ns // _BM,),
            in_specs=[
                pl.BlockSpec((_BM, d_feat), lambda i: (i, 0)),
                pl.BlockSpec((d_feat, n), lambda i: (0, 0)),
                pl.BlockSpec((_BM, 2), lambda i: (i, 0)),
                pl.BlockSpec((2, n), lambda i: (0, 0)),
            ],
            out_specs=[
                pl.BlockSpec((_BM, _KPAD), lambda i: (i, 0)),
                pl.BlockSpec((_BM, _KPAD * _LANES), lambda i: (i, 0)),
                pl.BlockSpec((_BM, _KPAD), lambda i: (i, 0)),
                pl.BlockSpec((_BM, _KPAD * _LANES), lambda i: (i, 0)),
            ],
            out_shape=[
                jax.ShapeDtypeStruct((ns, _KPAD), jnp.int32),
                jax.ShapeDtypeStruct((ns, _KPAD * _LANES), jnp.float32),
                jax.ShapeDtypeStruct((ns, _KPAD), jnp.int32),
                jax.ShapeDtypeStruct((ns, _KPAD * _LANES), jnp.float32),
            ],
        )(xs, xt, pcs, pct)

        idx_all = jnp.concatenate([sidx, pidx], axis=0).reshape(
            _NW, items // _CITEMS, _CITEMS)
        w_all = jnp.concatenate([sw, pw], axis=0).reshape(
            -1, _CITEMS // 8, 128)
        out_packed = pl.kernel(
            sc_fn,
            mesh=mesh,
            out_type=jax.ShapeDtypeStruct((q_slab * d_y // 128, 128),
                                          jnp.float32),
            scratch_types=[
                pltpu.VMEM((items // _CITEMS, _CITEMS), jnp.int32),
                pltpu.VMEM((_NBUF, _CITEMS, _YPAD), jnp.float32),
                pltpu.VMEM((_NBUF, _CITEMS // 8, 128), jnp.float32),
                pltpu.VMEM((qpw * _LANES // 128, 128), jnp.float32),
                pltpu.SemaphoreType.DMA((_NBUF,)),
            ],
        )(y_pad, idx_all, w_all)
        outs.append(out_packed.reshape(2, ns, d_y))
    return jnp.concatenate(outs, axis=1)


# speedup vs baseline: 1.1368x; 1.1368x over previous
import functools

import jax
from jax import lax
import jax.numpy as jnp
from jax.experimental import pallas as pl
from jax.experimental.pallas import tpu as pltpu
from jax.experimental.pallas import tpu_sc as plsc

_BM = 256
_KPAD = 5
_LANES = 16
_NW = 32
_CITEMS = 80
_YPAD = 128


def _topk_idx_w(s, k, kpad, exact_ties):
    bm, n = s.shape
    iota = (jax.lax.broadcasted_iota(jnp.int32, (bm, n), 1)
            if exact_ties else None)
    js, es = [], []
    v0 = None
    m = None
    for step in range(k):
        if exact_ties:
            t = s
        else:
            t = s if step == 0 else jnp.where(s < m, s, -jnp.inf)
        m = jnp.max(t, axis=1, keepdims=True)
        ji = jnp.argmax(t, axis=1)[:, None]
        if step == 0:
            v0 = m
            e = jnp.ones_like(m)
            z = jnp.ones_like(m)
        else:
            e = jnp.exp(m - v0)
            z = z + e
        js.append(ji.astype(jnp.int32))
        es.append(e)
        if exact_ties and step < k - 1:
            s = jnp.where(iota == ji, -jnp.inf, s)
    zi = 1.0 / z
    wcols = [jnp.broadcast_to(e * zi, (bm, _LANES)) for e in es]
    for _ in range(kpad - k):
        js.append(jnp.zeros_like(js[0]))
        wcols.append(jnp.zeros((bm, _LANES), jnp.float32))
    return jnp.concatenate(js, axis=1), jnp.concatenate(wcols, axis=1)


def _tc_body(x_ref, xt_ref, pc_ref, pct_ref,
             sidx_ref, sw_ref, pidx_ref, pw_ref, *, sim_k, spat_k):
    s = jnp.dot(x_ref[...], xt_ref[...], preferred_element_type=jnp.float32)
    sidx_ref[...], sw_ref[...] = _topk_idx_w(s, sim_k, _KPAD,
                                             exact_ties=False)

    pcb = pc_ref[...]
    pct = pct_ref[...]
    c2 = jnp.sum(pct * pct, axis=0, keepdims=True)
    c2b = jnp.sum(pcb * pcb, axis=1, keepdims=True)
    nd = -(c2b + c2
           - 2.0 * jnp.dot(pcb, pct, preferred_element_type=jnp.float32))
    pidx_ref[...], pw_ref[...] = _topk_idx_w(nd, spat_k, _KPAD,
                                             exact_ties=True)


_NBUF = 4


def _sc_gather_kernel(y_hbm, idx_hbm, w_hbm, out_hbm,
                      idx_v, rows_v, w_v, out_v, sems, *, qpw):
    wid = lax.axis_index("s") * 2 + lax.axis_index("c")
    items = qpw * _KPAD
    nchunks = items // _CITEMS
    qpc = _CITEMS // _KPAD
    wrows = _CITEMS // 8
    orows = qpc // 8

    pltpu.sync_copy(idx_hbm.at[wid], idx_v)

    def _fire(c, b):
        pltpu.async_copy(y_hbm.at[idx_v.at[c]], rows_v.at[b], sems.at[b])
        pltpu.async_copy(w_hbm.at[wid * nchunks + c], w_v.at[b], sems.at[b])

    def _wait(b):
        pltpu.make_async_copy(y_hbm.at[pl.ds(0, _CITEMS), :],
                              rows_v.at[b], sems.at[b]).wait()
        pltpu.make_async_copy(w_hbm.at[0], w_v.at[b], sems.at[b]).wait()

    def _compute(c, b):
        for qq in range(qpc):
            j0 = qq * _KPAD
            acc = None
            for k in range(_KPAD):
                j = j0 + k
                w = w_v[b, j // 8, pl.ds((j % 8) * _LANES, _LANES)]
                r = rows_v[b, j, pl.ds(0, _LANES)]
                acc = r * w if acc is None else acc + r * w
            out_v[c * orows + qq // 8,
                  pl.ds((qq % 8) * _LANES, _LANES)] = acc

    for b in range(_NBUF):
        _fire(b, b)

    def body(i, carry):
        for b in range(_NBUF):
            c = i * _NBUF + b
            _wait(b)
            _compute(c, b)
            _fire(c + _NBUF, b)
        return carry

    lax.fori_loop(0, nchunks // _NBUF - 1, body, 0)
    for b in range(_NBUF):
        c = nchunks - _NBUF + b
        _wait(b)
        _compute(c, b)

    opw = qpw * _LANES // 128
    pltpu.sync_copy(out_v, out_hbm.at[pl.ds(wid * opw, opw), :])


_NSLAB = 4


def kernel(x, y, patch_centers):
    n, d_feat = x.shape
    d_y = y.shape[1]
    xt = x.T
    pct = patch_centers.T
    y_pad = jnp.pad(y, ((0, 0), (0, _YPAD - d_y)))
    mesh = plsc.VectorSubcoreMesh(core_axis_name="c", subcore_axis_name="s")

    ns = n // _NSLAB
    q_slab = 2 * ns
    qpw = q_slab // _NW
    items = qpw * _KPAD
    tc_body = functools.partial(_tc_body, sim_k=5, spat_k=4)
    sc_fn = functools.partial(_sc_gather_kernel, qpw=qpw)

    outs = []
    for s in range(_NSLAB):
        xs = jax.lax.slice_in_dim(x, s * ns, (s + 1) * ns)
        pcs = jax.lax.slice_in_dim(patch_centers, s * ns, (s + 1) * ns)
        sidx, sw, pidx, pw = pl.pallas_call(
            tc_body,
            grid=(ns // _BM,),
            in_specs=[
                pl.BlockSpec((_BM, d_feat), lambda i: (i, 0)),
                pl.BlockSpec((d_feat, n), lambda i: (0, 0)),
                pl.BlockSpec((_BM, 2), lambda i: (i, 0)),
                pl.BlockSpec((2, n), lambda i: (0, 0)),
            ],
            out_specs=[
                pl.BlockSpec((_BM, _KPAD), lambda i: (i, 0)),
                pl.BlockSpec((_BM, _KPAD * _LANES), lambda i: (i, 0)),
                pl.BlockSpec((_BM, _KPAD), lambda i: (i, 0)),
                pl.BlockSpec((_BM, _KPAD * _LANES), lambda i: (i, 0)),
            ],
            out_shape=[
                jax.ShapeDtypeStruct((ns, _KPAD), jnp.int32),
                jax.ShapeDtypeStruct((ns, _KPAD * _LANES), jnp.float32),
                jax.ShapeDtypeStruct((ns, _KPAD), jnp.int32),
                jax.ShapeDtypeStruct((ns, _KPAD * _LANES), jnp.float32),
            ],
        )(xs, xt, pcs, pct)

        idx_all = jnp.concatenate([sidx, pidx], axis=0).reshape(
            _NW, items // _CITEMS, _CITEMS)
        w_all = jnp.concatenate([sw, pw], axis=0).reshape(
            -1, _CITEMS // 8, 128)
        out_packed = pl.kernel(
            sc_fn,
            mesh=mesh,
            out_type=jax.ShapeDtypeStruct((q_slab * d_y // 128, 128),
                                          jnp.float32),
            scratch_types=[
                pltpu.VMEM((items // _CITEMS, _CITEMS), jnp.int32),
                pltpu.VMEM((_NBUF, _CITEMS, _YPAD), jnp.float32),
                pltpu.VMEM((_NBUF, _CITEMS // 8, 128), jnp.float32),
                pltpu.VMEM((qpw * _LANES // 128, 128), jnp.float32),
                pltpu.SemaphoreType.DMA((_NBUF,)),
            ],
        )(y_pad, idx_all, w_all)
        outs.append(out_packed.reshape(2, ns, d_y))
    return jnp.concatenate(outs, axis=1)
